# Initial kernel scaffold; baseline (speedup 1.0000x reference)
import jax
import jax.numpy as jnp
from jax.experimental import pallas as pl


def kernel(node_fts, edges, edge_fts, W_self, W_neigh):
    raise NotImplementedError("write your pallas kernel here")


# speedup vs baseline: 4.0327x; 4.0327x over previous
import functools

import jax
import jax.numpy as jnp
from jax import lax
from jax.experimental import pallas as pl
from jax.experimental.pallas import tpu as pltpu
from jax.experimental.pallas import tpu_sc as plsc

N = 10000
E = 320000
D = 128
DH = D // 2

NC = 2
NS = 16
L = 16

K = 128
CHUNKS = 158
EPT = K * CHUNKS
E_PAD = EPT * NS
N_PAD = 10240
ROWS_PT = N_PAD // NS
DUMMY_ROW = 10100
RB = 128


def _mm_body(x_ref, wn0_ref, wn1_ref, ws_ref, h0_ref, h1_ref, s_ref):
    x = x_ref[...]
    dn = (((1,), (1,)), ((), ()))
    h0_ref[...] = lax.dot_general(x, wn0_ref[...], dn,
                                  preferred_element_type=jnp.float32)
    h1_ref[...] = lax.dot_general(x, wn1_ref[...], dn,
                                  preferred_element_type=jnp.float32)
    s_ref[...] = lax.dot_general(x, ws_ref[...], dn,
                                 preferred_element_type=jnp.float32)


def _matmuls(x, wn, ws):
    blk = 1000
    grid = N // blk
    return pl.pallas_call(
        _mm_body,
        grid=(grid,),
        in_specs=[
            pl.BlockSpec((blk, D), lambda i: (i, 0)),
            pl.BlockSpec((DH, D), lambda i: (0, 0)),
            pl.BlockSpec((DH, D), lambda i: (0, 0)),
            pl.BlockSpec((D, D), lambda i: (0, 0)),
        ],
        out_specs=[
            pl.BlockSpec((blk, DH), lambda i: (i, 0)),
            pl.BlockSpec((blk, DH), lambda i: (i, 0)),
            pl.BlockSpec((blk, D), lambda i: (i, 0)),
        ],
        out_shape=[
            jax.ShapeDtypeStruct((N, DH), jnp.float32),
            jax.ShapeDtypeStruct((N, DH), jnp.float32),
            jax.ShapeDtypeStruct((N, D), jnp.float32),
        ],
    )(x, wn[:DH], wn[DH:], ws)


def _sc_body(h0_hbm, h1_hbm, src_hbm, dst_hbm, ps_hbm, pc_hbm,
             dsti, srci, rows, ones_b, zt, zc, acc, cnt):
    c = lax.axis_index("c")
    s = lax.axis_index("s")

    def init_row(r, carry):
        for j in range(DH // L):
            zt[r, pl.ds(j * L, L)] = jnp.zeros((L,), jnp.float32)
        zc[r, :] = jnp.zeros((L,), jnp.float32)
        ones_b[r, :] = jnp.ones((L,), jnp.float32)
        return carry

    lax.fori_loop(0, RB, init_row, 0)

    row0 = s * ROWS_PT

    def zero_blk(j, carry):
        pltpu.sync_copy(zt, acc.at[pl.ds(row0 + j * RB, RB), :])
        pltpu.sync_copy(zc, cnt.at[pl.ds(row0 + j * RB, RB), :])
        return carry

    lax.fori_loop(0, ROWS_PT // RB, zero_blk, 0)
    plsc.subcore_barrier()

    ebase = s * EPT

    @pl.when(c == 0)
    def _core0():
        def edge_chunk(i, carry):
            base = ebase + i * K
            pltpu.sync_copy(dst_hbm.at[pl.ds(base, K)], dsti)
            pltpu.sync_copy(src_hbm.at[pl.ds(base, K)], srci)
            pltpu.sync_copy(h0_hbm.at[dsti], rows)
            pltpu.sync_copy(rows, acc.at[srci], add=True)
            pltpu.sync_copy(ones_b, cnt.at[srci], add=True)
            return carry

        lax.fori_loop(0, CHUNKS, edge_chunk, 0)

    @pl.when(c == 1)
    def _core1():
        def edge_chunk(i, carry):
            base = ebase + i * K
            pltpu.sync_copy(dst_hbm.at[pl.ds(base, K)], dsti)
            pltpu.sync_copy(src_hbm.at[pl.ds(base, K)], srci)
            pltpu.sync_copy(h1_hbm.at[dsti], rows)
            pltpu.sync_copy(rows, acc.at[srci], add=True)
            return carry

        lax.fori_loop(0, CHUNKS, edge_chunk, 0)

    plsc.subcore_barrier()

    def out_blk(j, carry):
        r = row0 + j * RB
        pltpu.sync_copy(acc.at[pl.ds(r, RB), :], zt)
        pltpu.sync_copy(zt, ps_hbm.at[c, pl.ds(r, RB), :])
        return carry

    lax.fori_loop(0, ROWS_PT // RB, out_blk, 0)

    @pl.when(c == 0)
    def _cnt_out():
        def cnt_blk(j, carry):
            r = row0 + j * RB
            pltpu.sync_copy(cnt.at[pl.ds(r, RB), :], zc)
            pltpu.sync_copy(zc, pc_hbm.at[pl.ds(r, RB), :])
            return carry

        lax.fori_loop(0, ROWS_PT // RB, cnt_blk, 0)


_sc_call = functools.partial(
    pl.kernel,
    out_type=[
        jax.ShapeDtypeStruct((NC, N_PAD, DH), jnp.float32),
        jax.ShapeDtypeStruct((N_PAD, L), jnp.float32),
    ],
    mesh=plsc.VectorSubcoreMesh(core_axis_name="c", subcore_axis_name="s"),
    compiler_params=pltpu.CompilerParams(use_tc_tiling_on_sc=False),
    scratch_types=[
        pltpu.VMEM((K,), jnp.int32),
        pltpu.VMEM((K,), jnp.int32),
        pltpu.VMEM((K, DH), jnp.float32),
        pltpu.VMEM((K, L), jnp.float32),
        pltpu.VMEM((RB, DH), jnp.float32),
        pltpu.VMEM((RB, L), jnp.float32),
        pltpu.VMEM_SHARED((N_PAD, DH), jnp.float32),
        pltpu.VMEM_SHARED((N_PAD, L), jnp.float32),
    ],
)(_sc_body)


def _fin_body(s_ref, ps_ref, pc_ref, o_ref):
    sums = jnp.concatenate([ps_ref[0], ps_ref[1]], axis=1)
    cnts = pc_ref[:, 0:1]
    agg = sums / jnp.maximum(cnts, 1.0)
    neigh = jnp.where(agg >= 0, agg, 0.2 * agg)
    out = s_ref[...] + neigh
    out = jnp.where(out >= 0, out, 0.2 * out)
    nrm = jnp.sqrt(jnp.sum(out * out, axis=1, keepdims=True))
    o_ref[...] = out / jnp.maximum(nrm, 1e-12)


def _finalize(s, ps, pc):
    blk = 1000
    grid = N // blk
    return pl.pallas_call(
        _fin_body,
        grid=(grid,),
        in_specs=[
            pl.BlockSpec((blk, D), lambda i: (i, 0)),
            pl.BlockSpec((NC, blk, DH), lambda i: (0, i, 0)),
            pl.BlockSpec((blk, L), lambda i: (i, 0)),
        ],
        out_specs=pl.BlockSpec((blk, D), lambda i: (i, 0)),
        out_shape=jax.ShapeDtypeStruct((N, D), jnp.float32),
    )(s, ps, pc)


@jax.jit
def kernel(node_fts, edges, edge_fts, W_self, W_neigh):
    del edge_fts
    h0, h1, s = _matmuls(node_fts, W_neigh, W_self)
    pad = E_PAD - E
    src_p = jnp.concatenate(
        [edges[0], jnp.full((pad,), DUMMY_ROW, jnp.int32)])
    dst_p = jnp.concatenate([edges[1], jnp.zeros((pad,), jnp.int32)])
    ps, pc = _sc_call(h0, h1, src_p, dst_p)
    return _finalize(s, ps, pc)


# speedup vs baseline: 5.5856x; 1.3851x over previous
import functools

import jax
import jax.numpy as jnp
from jax import lax
from jax.experimental import pallas as pl
from jax.experimental.pallas import tpu as pltpu
from jax.experimental.pallas import tpu_sc as plsc

N = 10000
E = 320000
D = 128
DH = D // 2

NC = 2
NS = 16
L = 16

K = 128
CHUNKS = 160
PCH = CHUNKS // 2
EPT = K * CHUNKS
E_PAD = EPT * NS
NROW2D = E_PAD // K
N_PAD = 10240
ROWS_PT = N_PAD // NS
DUMMY_ROW = 10100
RB = 128
NBUF = 4


def _mm_body(x_ref, wn_ref, h_ref):
    dn = (((1,), (1,)), ((), ()))
    h_ref[0] = lax.dot_general(x_ref[...], wn_ref[0], dn,
                               preferred_element_type=jnp.float32)


def _matmul_h(x, wn2):
    blk = 1000
    return pl.pallas_call(
        _mm_body,
        grid=(NC, N // blk),
        in_specs=[
            pl.BlockSpec((blk, D), lambda g, i: (i, 0)),
            pl.BlockSpec((1, DH, D), lambda g, i: (g, 0, 0)),
        ],
        out_specs=pl.BlockSpec((1, blk, DH), lambda g, i: (g, i, 0)),
        out_shape=jax.ShapeDtypeStruct((NC, N, DH), jnp.float32),
    )(x, wn2)


def _sc_body(h_hbm, src_hbm, dst_hbm, ps_hbm, pc_hbm,
             src_t, dst_t, r0, r1, r2, r3, ones_b, zt, zc, acc, cnt,
             gs0, gs1, gs2, gs3, ss0, ss1, ss2, ss3, csem):
    c = lax.axis_index("c")
    s = lax.axis_index("s")
    rows = (r0, r1, r2, r3)
    gsem = (gs0, gs1, gs2, gs3)
    ssem = (ss0, ss1, ss2, ss3)
    row0_2d = s * CHUNKS

    def init_row(r, carry):
        for j in range(DH // L):
            zt[r, pl.ds(j * L, L)] = jnp.zeros((L,), jnp.float32)
        zc[r, :] = jnp.zeros((L,), jnp.float32)
        ones_b[r, :] = jnp.ones((L,), jnp.float32)
        return carry

    lax.fori_loop(0, RB, init_row, 0)

    arow0 = s * ROWS_PT

    def zero_blk(j, carry):
        pltpu.sync_copy(zt, acc.at[pl.ds(arow0 + j * RB, RB), :])
        pltpu.sync_copy(zc, cnt.at[pl.ds(arow0 + j * RB, RB), :])
        return carry

    lax.fori_loop(0, ROWS_PT // RB, zero_blk, 0)
    plsc.subcore_barrier()

    def run_phase(base, count_core):
        pltpu.sync_copy(src_hbm.at[pl.ds(row0_2d + base, PCH), :], src_t)
        pltpu.sync_copy(dst_hbm.at[c, pl.ds(row0_2d + base, PCH), :], dst_t)

        def g_issue(i, b):
            pltpu.async_copy(h_hbm.at[dst_t.at[i]], rows[b], gsem[b])

        def g_wait(i, b):
            pltpu.make_async_copy(
                h_hbm.at[dst_t.at[i]], rows[b], gsem[b]).wait()

        def s_issue(i, b):
            pltpu.async_copy(rows[b], acc.at[src_t.at[i]], ssem[b], add=True)

            @pl.when(c == count_core)
            def _():
                pltpu.async_copy(ones_b, cnt.at[src_t.at[i]], csem, add=True)

        def s_wait(i, b):
            pltpu.make_async_copy(
                rows[b], acc.at[src_t.at[i]], ssem[b]).wait()

        g_issue(0, 0)
        g_issue(1, 1)
        g_issue(2, 2)
        g_wait(0, 0)
        s_issue(0, 0)
        g_issue(3, 3)
        g_wait(1, 1)
        s_issue(1, 1)

        def main(k, carry):
            for boff in range(NBUF):
                i = NBUF * k + boff
                s_wait(i - NBUF, boff)
                g_issue(i, boff)
                j = i - 2
                bj = (boff + 2) % NBUF
                g_wait(j, bj)
                s_issue(j, bj)
            return carry

        lax.fori_loop(1, PCH // NBUF, main, 0)

        g_wait(PCH - 2, 2)
        s_issue(PCH - 2, 2)
        g_wait(PCH - 1, 3)
        s_issue(PCH - 1, 3)
        s_wait(PCH - 4, 0)
        s_wait(PCH - 3, 1)
        s_wait(PCH - 2, 2)
        s_wait(PCH - 1, 3)

        @pl.when(c == count_core)
        def _():
            def cnt_drain(i, carry):
                pltpu.make_async_copy(
                    ones_b, cnt.at[src_t.at[0]], csem).wait()
                return carry

            lax.fori_loop(0, PCH, cnt_drain, 0)

    run_phase(0, 0)
    run_phase(PCH, 1)
    plsc.subcore_barrier()

    def out_blk(j, carry):
        r = arow0 + j * RB
        pltpu.sync_copy(acc.at[pl.ds(r, RB), :], zt)
        pltpu.sync_copy(zt, ps_hbm.at[c, pl.ds(r, RB), :])
        pltpu.sync_copy(cnt.at[pl.ds(r, RB), :], zc)
        pltpu.sync_copy(zc, pc_hbm.at[c, pl.ds(r, RB), :])
        return carry

    lax.fori_loop(0, ROWS_PT // RB, out_blk, 0)


_sc_call = functools.partial(
    pl.kernel,
    out_type=[
        jax.ShapeDtypeStruct((NC, N_PAD, DH), jnp.float32),
        jax.ShapeDtypeStruct((NC, N_PAD, L), jnp.float32),
    ],
    mesh=plsc.VectorSubcoreMesh(core_axis_name="c", subcore_axis_name="s"),
    compiler_params=pltpu.CompilerParams(use_tc_tiling_on_sc=False),
    scratch_types=[
        pltpu.VMEM((PCH, K), jnp.int32),
        pltpu.VMEM((PCH, K), jnp.int32),
        pltpu.VMEM((K, DH), jnp.float32),
        pltpu.VMEM((K, DH), jnp.float32),
        pltpu.VMEM((K, DH), jnp.float32),
        pltpu.VMEM((K, DH), jnp.float32),
        pltpu.VMEM((K, L), jnp.float32),
        pltpu.VMEM((RB, DH), jnp.float32),
        pltpu.VMEM((RB, L), jnp.float32),
        pltpu.VMEM_SHARED((N_PAD, DH), jnp.float32),
        pltpu.VMEM_SHARED((N_PAD, L), jnp.float32),
        pltpu.SemaphoreType.DMA,
        pltpu.SemaphoreType.DMA,
        pltpu.SemaphoreType.DMA,
        pltpu.SemaphoreType.DMA,
        pltpu.SemaphoreType.DMA,
        pltpu.SemaphoreType.DMA,
        pltpu.SemaphoreType.DMA,
        pltpu.SemaphoreType.DMA,
        pltpu.SemaphoreType.DMA,
    ],
)(_sc_body)


def _fin_body(x_ref, ws_ref, ps_ref, pc_ref, o_ref):
    dn = (((1,), (1,)), ((), ()))
    sf = lax.dot_general(x_ref[...], ws_ref[...], dn,
                         preferred_element_type=jnp.float32)
    sums = jnp.concatenate([ps_ref[0], ps_ref[1]], axis=1)
    cnts = pc_ref[0, :, 0:1] + pc_ref[1, :, 0:1]
    agg = sums / jnp.maximum(cnts, 1.0)
    neigh = jnp.where(agg >= 0, agg, 0.2 * agg)
    out = sf + neigh
    out = jnp.where(out >= 0, out, 0.2 * out)
    nrm = jnp.sqrt(jnp.sum(out * out, axis=1, keepdims=True))
    o_ref[...] = out / jnp.maximum(nrm, 1e-12)


def _finalize(x, ws, ps, pc):
    blk = 1000
    grid = N // blk
    return pl.pallas_call(
        _fin_body,
        grid=(grid,),
        in_specs=[
            pl.BlockSpec((blk, D), lambda i: (i, 0)),
            pl.BlockSpec((D, D), lambda i: (0, 0)),
            pl.BlockSpec((NC, blk, DH), lambda i: (0, i, 0)),
            pl.BlockSpec((NC, blk, L), lambda i: (0, i, 0)),
        ],
        out_specs=pl.BlockSpec((blk, D), lambda i: (i, 0)),
        out_shape=jax.ShapeDtypeStruct((N, D), jnp.float32),
    )(x, ws, ps, pc)


@jax.jit
def kernel(node_fts, edges, edge_fts, W_self, W_neigh):
    del edge_fts
    h = _matmul_h(node_fts, W_neigh.reshape(NC, DH, D))
    h_flat = h.reshape(NC * N, DH)
    pad = E_PAD - E
    src_p = jnp.concatenate(
        [edges[0], jnp.full((pad,), DUMMY_ROW, jnp.int32)])
    dst_p = jnp.concatenate([edges[1], jnp.zeros((pad,), jnp.int32)])
    src2d = src_p.reshape(NROW2D, K)
    dst3d = jnp.stack([dst_p, dst_p + N]).reshape(NC, NROW2D, K)
    ps, pc = _sc_call(h_flat, src2d, dst3d)
    return _finalize(node_fts, W_self, ps, pc)
